# Initial kernel scaffold; baseline (speedup 1.0000x reference)
import jax
import jax.numpy as jnp
from jax.experimental import pallas as pl


def kernel(logits, top_k):
    raise NotImplementedError("write your pallas kernel here")


# speedup vs baseline: 7.9057x; 7.9057x over previous
import functools
import math

import jax
import jax.numpy as jnp
from jax.experimental import pallas as pl
from jax.experimental.pallas import tpu as pltpu

_NB = 1024
_CAP = 64
_K = 50
_ROWBLK = 8
_TINY = float(jnp.finfo(jnp.float32).tiny)


def _threefry_bits(x1):
    k1 = jnp.uint32(0)
    k2 = jnp.uint32(1)
    ks2 = jnp.uint32(0x1BD11BDA) ^ k1 ^ k2
    ks = (k1, k2, ks2)
    rot = ((13, 15, 26, 6), (17, 29, 16, 24))
    x0 = jnp.zeros_like(x1) + k1
    x1 = x1 + k2
    for i in range(5):
        for r in rot[i % 2]:
            x0 = x0 + x1
            x1 = (x1 << jnp.uint32(r)) | (x1 >> jnp.uint32(32 - r))
            x1 = x1 ^ x0
        x0 = x0 + ks[(i + 1) % 3]
        x1 = x1 + ks[(i + 2) % 3] + jnp.uint32(i + 1)
    return x0 ^ x1


def _gumbel_from_bits(bits):
    fb = (bits >> jnp.uint32(9)) | jnp.uint32(0x3F800000)
    f = jax.lax.bitcast_convert_type(fb, jnp.float32) - jnp.float32(1.0)
    tiny = jnp.float32(_TINY)
    u = jax.lax.max(tiny, f * (jnp.float32(1.0) - tiny) + tiny)
    return -jnp.log(-jnp.log(u))


def _pass_a_kernel(x_ref, probs_ref, bmax_ref, idx_ref, *, ncb, ncols, cap):
    b = pl.program_id(1)
    last = ncb - 1
    rem = ncols - last * _NB
    x = x_ref[...]
    colio = jax.lax.broadcasted_iota(jnp.int32, x.shape, 1)
    x = jnp.where((b == last) & (colio >= rem), -jnp.inf, x)
    probs_ref[...] = jnp.zeros_like(x)
    cio = jax.lax.broadcasted_iota(jnp.int32, bmax_ref.shape, 1)
    m_here = jnp.max(x, axis=1, keepdims=True)
    bmax_ref[...] = jnp.where(cio == b, m_here, bmax_ref[...])

    @pl.when(b == last)
    def _():
        bm = bmax_ref[...]
        rows = bm.shape[0]
        capio = jax.lax.broadcasted_iota(jnp.int32, (rows, cap), 1)

        def body(t, carry):
            bm, acc = carry
            m = jnp.max(bm, axis=1, keepdims=True)
            pos = jnp.min(
                jnp.where(bm == m, cio, jnp.int32(1 << 30)), axis=1, keepdims=True
            )
            acc = jnp.where(capio == t, pos, acc)
            bm = jnp.where(cio == pos, -jnp.inf, bm)
            return bm, acc

        _, acc = jax.lax.fori_loop(
            0, cap, body, (bm, jnp.zeros((rows, cap), jnp.int32))
        )
        idx_ref[...] = acc


def _pass_b_kernel(
    idx_sref, x_ref, out_ref, vals_ref, gcol_ref, vtmp_ref, *, ncols, cap
):
    r = pl.program_id(0)
    s = pl.program_id(1)
    blk = idx_sref[r, s]
    colio = jax.lax.broadcasted_iota(jnp.int32, (1, _NB), 1)
    gcol = blk * _NB + colio
    x = jnp.where(gcol >= ncols, -jnp.inf, x_ref[0])
    vals_ref[pl.ds(s, 1), :] = x
    gcol_ref[pl.ds(s, 1), :] = gcol

    @pl.when(s == cap - 1)
    def _():
        vals = vals_ref[...]
        gc = gcol_ref[...]
        rowmax = jnp.max(vals)
        vtmp_ref[...] = vals

        def tbody(i, carry):
            remk, thresh = carry
            v = vtmp_ref[...]
            m = jnp.max(v)
            c = jnp.sum((v == m).astype(jnp.int32))
            active = remk > 0
            hit = active & (c >= remk)
            thresh = jnp.where(hit, m, thresh)
            newrem = jnp.where(active & ~hit, remk - c, jnp.int32(0))
            vtmp_ref[...] = jnp.where(active & ~hit & (v == m), -jnp.inf, v)
            return newrem, thresh

        _, thresh = jax.lax.fori_loop(
            0, _K, tbody, (jnp.int32(_K), -jnp.inf), unroll=False
        )

        keep = vals >= thresh
        z = jnp.sum(jnp.where(keep, jnp.exp(vals - rowmax), 0.0))

        p = (r * ncols + gc).astype(jnp.uint32)
        g = _gumbel_from_bits(_threefry_bits(p))
        cand = jnp.where(keep, vals + g, -jnp.inf)
        m2 = jnp.max(cand)
        wcol = jnp.min(jnp.where(cand == m2, gc, jnp.int32(1 << 30)))

        oio = jax.lax.broadcasted_iota(jnp.int32, (1, 1, 128), 2)
        vec = jnp.where(
            oio == 0,
            thresh,
            jnp.where(
                oio == 1,
                z,
                jnp.where(
                    oio == 2, rowmax, jnp.where(oio == 3, wcol.astype(jnp.float32), 0.0)
                ),
            ),
        )
        out_ref[...] = vec


def _pass_c_kernel(idx_sref, zeros_ref, x_ref, par_ref, out_ref):
    del idx_sref, zeros_ref
    x = x_ref[0]
    thresh = par_ref[0, 0, 0]
    z = par_ref[0, 0, 1]
    rowmax = par_ref[0, 0, 2]
    out_ref[0] = jnp.where(x >= thresh, jnp.exp(x - rowmax) / z, 0.0)


def kernel(logits, top_k):
    del top_k
    nrows, ncols = logits.shape
    ncb = math.ceil(ncols / _NB)
    cap = min(_CAP, ncb)
    rblk = min(_ROWBLK, nrows)

    probs0, _, topidx = pl.pallas_call(
        functools.partial(_pass_a_kernel, ncb=ncb, ncols=ncols, cap=cap),
        grid=(nrows // rblk, ncb),
        in_specs=[pl.BlockSpec((rblk, _NB), lambda r, b: (r, b))],
        out_specs=[
            pl.BlockSpec((rblk, _NB), lambda r, b: (r, b)),
            pl.BlockSpec((rblk, ncb), lambda r, b: (r, 0)),
            pl.BlockSpec((rblk, cap), lambda r, b: (r, 0)),
        ],
        out_shape=[
            jax.ShapeDtypeStruct((nrows, ncols), jnp.float32),
            jax.ShapeDtypeStruct((nrows, ncb), jnp.float32),
            jax.ShapeDtypeStruct((nrows, cap), jnp.int32),
        ],
    )(logits)

    logits3 = logits.reshape(nrows, 1, ncols)

    params = pl.pallas_call(
        functools.partial(_pass_b_kernel, ncols=ncols, cap=cap),
        grid_spec=pltpu.PrefetchScalarGridSpec(
            num_scalar_prefetch=1,
            grid=(nrows, cap),
            in_specs=[
                pl.BlockSpec((1, 1, _NB), lambda r, s, idx: (r, 0, idx[r, s])),
            ],
            out_specs=pl.BlockSpec((1, 1, 128), lambda r, s, idx: (r, 0, 0)),
            scratch_shapes=[
                pltpu.VMEM((cap, _NB), jnp.float32),
                pltpu.VMEM((cap, _NB), jnp.int32),
                pltpu.VMEM((cap, _NB), jnp.float32),
            ],
        ),
        out_shape=jax.ShapeDtypeStruct((nrows, 1, 128), jnp.float32),
    )(topidx, logits3)

    probs3 = pl.pallas_call(
        _pass_c_kernel,
        grid_spec=pltpu.PrefetchScalarGridSpec(
            num_scalar_prefetch=1,
            grid=(nrows, cap),
            in_specs=[
                pl.BlockSpec(memory_space=pl.ANY),
                pl.BlockSpec((1, 1, _NB), lambda r, s, idx: (r, 0, idx[r, s])),
                pl.BlockSpec((1, 1, 128), lambda r, s, idx: (r, 0, 0)),
            ],
            out_specs=pl.BlockSpec((1, 1, _NB), lambda r, s, idx: (r, 0, idx[r, s])),
        ),
        out_shape=jax.ShapeDtypeStruct((nrows, 1, ncols), jnp.float32),
        input_output_aliases={1: 0},
    )(topidx, probs0.reshape(nrows, 1, ncols), logits3, params)

    idx_next = params[:, 0, 3].astype(jnp.int32)
    return probs3.reshape(nrows, ncols), idx_next


# speedup vs baseline: 22.8366x; 2.8886x over previous
import functools
import math

import jax
import jax.numpy as jnp
from jax.experimental import pallas as pl
from jax.experimental.pallas import tpu as pltpu

_NB = 1000
_CAP = 64
_K = 50
_TINY = float(jnp.finfo(jnp.float32).tiny)


def _threefry_bits(x1):
    k1 = jnp.uint32(0)
    k2 = jnp.uint32(1)
    ks2 = jnp.uint32(0x1BD11BDA) ^ k1 ^ k2
    ks = (k1, k2, ks2)
    rot = ((13, 15, 26, 6), (17, 29, 16, 24))
    x0 = jnp.zeros_like(x1) + k1
    x1 = x1 + k2
    for i in range(5):
        for r in rot[i % 2]:
            x0 = x0 + x1
            x1 = (x1 << jnp.uint32(r)) | (x1 >> jnp.uint32(32 - r))
            x1 = x1 ^ x0
        x0 = x0 + ks[(i + 1) % 3]
        x1 = x1 + ks[(i + 2) % 3] + jnp.uint32(i + 1)
    return x0 ^ x1


def _gumbel_from_bits(bits):
    fb = (bits >> jnp.uint32(9)) | jnp.uint32(0x3F800000)
    f = jax.lax.bitcast_convert_type(fb, jnp.float32) - jnp.float32(1.0)
    tiny = jnp.float32(_TINY)
    u = jax.lax.max(tiny, f * (jnp.float32(1.0) - tiny) + tiny)
    return -jnp.log(-jnp.log(u))


def _pass_a_kernel(x_ref, bmax_ref):
    bmax_ref[...] = jnp.max(x_ref[...], axis=2, keepdims=True)


def _pass_a2_kernel(bmax_ref, idx_ref, *, cap):
    bm = bmax_ref[...]
    rows, ncb = bm.shape
    cio = jax.lax.broadcasted_iota(jnp.int32, bm.shape, 1)
    capio = jax.lax.broadcasted_iota(jnp.int32, (rows, cap), 1)

    def body(t, carry):
        bm, acc = carry
        m = jnp.max(bm, axis=1, keepdims=True)
        pos = jnp.min(
            jnp.where(bm == m, cio, jnp.int32(1 << 30)), axis=1, keepdims=True
        )
        acc = jnp.where(capio == t, pos, acc)
        bm = jnp.where(cio == pos, -jnp.inf, bm)
        return bm, acc

    _, acc = jax.lax.fori_loop(0, cap, body, (bm, jnp.zeros((rows, cap), jnp.int32)))
    idx_ref[...] = acc


def _pass_b_kernel(idx_sref, *refs, ncols, cap):
    x_refs = refs[:cap]
    out_ref = refs[cap]
    vals_ref, gcol_ref = refs[cap + 1], refs[cap + 2]
    r = pl.program_id(0)
    lane = jax.lax.broadcasted_iota(jnp.int32, (1, _NB), 1)
    for j in range(cap):
        blk = idx_sref[r, j]
        vals_ref[pl.ds(j, 1), :] = x_refs[j][0, 0]
        gcol_ref[pl.ds(j, 1), :] = blk * _NB + lane

    vals = vals_ref[...]
    gc = gcol_ref[...]
    rowmax = jnp.max(vals)

    b = jax.lax.bitcast_convert_type(vals, jnp.uint32)
    hi = jnp.uint32(0x80000000)
    s = jnp.where((b & hi) != 0, ~b, b | hi)

    def bbody(i, t):
        t_try = t | (hi >> i.astype(jnp.uint32))
        cnt = jnp.sum((s >= t_try).astype(jnp.int32))
        return jnp.where(cnt >= _K, t_try, t)

    t_bits = jax.lax.fori_loop(0, 32, bbody, jnp.uint32(0))
    tb = jnp.where((t_bits & hi) != 0, t_bits & ~hi, ~t_bits)
    thresh = jax.lax.bitcast_convert_type(tb, jnp.float32)

    keep = vals >= thresh
    z = jnp.sum(jnp.where(keep, jnp.exp(vals - rowmax), 0.0))

    p = (r * ncols + gc).astype(jnp.uint32)
    g = _gumbel_from_bits(_threefry_bits(p))
    cand = jnp.where(keep, vals + g, -jnp.inf)
    m2 = jnp.max(cand)
    wcol = jnp.min(jnp.where(cand == m2, gc, jnp.int32(1 << 30)))

    oio = jax.lax.broadcasted_iota(jnp.int32, (1, 1, 128), 2)
    vec = jnp.where(
        oio == 0,
        thresh,
        jnp.where(
            oio == 1,
            z,
            jnp.where(
                oio == 2, rowmax, jnp.where(oio == 3, wcol.astype(jnp.float32), 0.0)
            ),
        ),
    )
    out_ref[...] = vec


def _pass_c_kernel(x_ref, par_ref, out_ref):
    x = x_ref[...]
    thresh = par_ref[0, 0, 0]
    z = par_ref[0, 0, 1]
    rowmax = par_ref[0, 0, 2]
    out_ref[...] = jnp.where(x >= thresh, jnp.exp(x - rowmax) / z, 0.0)


def kernel(logits, top_k):
    del top_k
    nrows, ncols = logits.shape
    ncb = math.ceil(ncols / _NB)
    cap = min(_CAP, ncb)
    x3 = logits.reshape(nrows, ncb, _NB)

    rq = 200 if (ncb % 200 == 0) else ncb
    bmax = pl.pallas_call(
        _pass_a_kernel,
        grid=(nrows, ncb // rq),
        in_specs=[pl.BlockSpec((1, rq, _NB), lambda r, q: (r, q, 0))],
        out_specs=pl.BlockSpec((1, rq, 1), lambda r, q: (r, q, 0)),
        out_shape=jax.ShapeDtypeStruct((nrows, ncb, 1), jnp.float32),
    )(x3)

    topidx = pl.pallas_call(
        functools.partial(_pass_a2_kernel, cap=cap),
        grid=(1,),
        in_specs=[pl.BlockSpec((nrows, ncb), lambda i: (0, 0))],
        out_specs=pl.BlockSpec((nrows, cap), lambda i: (0, 0)),
        out_shape=jax.ShapeDtypeStruct((nrows, cap), jnp.int32),
    )(bmax.reshape(nrows, ncb))

    params = pl.pallas_call(
        functools.partial(_pass_b_kernel, ncols=ncols, cap=cap),
        grid_spec=pltpu.PrefetchScalarGridSpec(
            num_scalar_prefetch=1,
            grid=(nrows,),
            in_specs=[
                pl.BlockSpec((1, 1, 1, _NB), functools.partial(
                    lambda j, r, idx: (r, idx[r, j], 0, 0), j))
                for j in range(cap)
            ],
            out_specs=pl.BlockSpec((1, 1, 128), lambda r, idx: (r, 0, 0)),
            scratch_shapes=[
                pltpu.VMEM((cap, _NB), jnp.float32),
                pltpu.VMEM((cap, _NB), jnp.int32),
            ],
        ),
        out_shape=jax.ShapeDtypeStruct((nrows, 1, 128), jnp.float32),
    )(topidx, *([x3.reshape(nrows, ncb, 1, _NB)] * cap))

    probs3 = pl.pallas_call(
        _pass_c_kernel,
        grid=(nrows, ncb // rq),
        in_specs=[
            pl.BlockSpec((1, rq, _NB), lambda r, q: (r, q, 0)),
            pl.BlockSpec((1, 1, 128), lambda r, q: (r, 0, 0)),
        ],
        out_specs=pl.BlockSpec((1, rq, _NB), lambda r, q: (r, q, 0)),
        out_shape=jax.ShapeDtypeStruct((nrows, ncb, _NB), jnp.float32),
    )(x3, params)

    idx_next = params[:, 0, 3].astype(jnp.int32)
    return probs3.reshape(nrows, ncols), idx_next


# speedup vs baseline: 68.7342x; 3.0098x over previous
import functools
import math

import jax
import jax.numpy as jnp
from jax.experimental import pallas as pl
from jax.experimental.pallas import tpu as pltpu

_NB = 1024
_W = 65536
_CAP = 64
_K = 50
_TINY = float(jnp.finfo(jnp.float32).tiny)


def _threefry_bits(x1):
    k1 = jnp.uint32(0)
    k2 = jnp.uint32(1)
    ks2 = jnp.uint32(0x1BD11BDA) ^ k1 ^ k2
    ks = (k1, k2, ks2)
    rot = ((13, 15, 26, 6), (17, 29, 16, 24))
    x0 = jnp.zeros_like(x1) + k1
    x1 = x1 + k2
    for i in range(5):
        for r in rot[i % 2]:
            x0 = x0 + x1
            x1 = (x1 << jnp.uint32(r)) | (x1 >> jnp.uint32(32 - r))
            x1 = x1 ^ x0
        x0 = x0 + ks[(i + 1) % 3]
        x1 = x1 + ks[(i + 2) % 3] + jnp.uint32(i + 1)
    return x0 ^ x1


def _gumbel_from_bits(bits):
    fb = (bits >> jnp.uint32(9)) | jnp.uint32(0x3F800000)
    f = jax.lax.bitcast_convert_type(fb, jnp.float32) - jnp.float32(1.0)
    tiny = jnp.float32(_TINY)
    u = jax.lax.max(tiny, f * (jnp.float32(1.0) - tiny) + tiny)
    return -jnp.log(-jnp.log(u))


def _pass_a_kernel(x_ref, bmax_ref, *, ncols, ncb, nsub):
    c = pl.program_id(1)
    x = x_ref[...]
    rows = x.shape[0]
    cio = jax.lax.broadcasted_iota(jnp.int32, (rows, ncb), 1)
    lane = jax.lax.broadcasted_iota(jnp.int32, (rows, _NB), 1)
    acc = bmax_ref[...]
    for i in range(nsub):
        sub = x[:, i * _NB : (i + 1) * _NB]
        gcol = c * _W + i * _NB + lane
        sub = jnp.where(gcol >= ncols, -jnp.inf, sub)
        m = jnp.max(sub, axis=1, keepdims=True)
        acc = jnp.where(cio == (c * nsub + i), m, acc)
    bmax_ref[...] = acc


def _pass_a2_kernel(bmax_ref, idx_ref, *, cap):
    bm = bmax_ref[...]
    rows, ncb = bm.shape
    cio = jax.lax.broadcasted_iota(jnp.int32, bm.shape, 1)
    capio = jax.lax.broadcasted_iota(jnp.int32, (rows, cap), 1)

    def body(t, carry):
        bm, acc = carry
        m = jnp.max(bm, axis=1, keepdims=True)
        pos = jnp.min(
            jnp.where(bm == m, cio, jnp.int32(1 << 30)), axis=1, keepdims=True
        )
        acc = jnp.where(capio == t, pos, acc)
        bm = jnp.where(cio == pos, -jnp.inf, bm)
        return bm, acc

    _, acc = jax.lax.fori_loop(0, cap, body, (bm, jnp.zeros((rows, cap), jnp.int32)))
    idx_ref[...] = acc


def _pass_b_kernel(
    idx_sref, x_hbm, tail_ref, out_ref, slabs_ref, gcol_ref, sems, *, nrows, ncols, cap
):
    r = pl.program_id(0)
    afloor = ((ncols - _NB) // 128) * 128
    rbase = pl.multiple_of((r // 8) * 8, 8)
    copies = []
    for j in range(cap):
        blk = idx_sref[r, j]
        start = pl.multiple_of(jnp.minimum(blk * _NB, afloor), 128)
        cp = pltpu.make_async_copy(
            x_hbm.at[pl.ds(rbase, 8), pl.ds(start, _NB)],
            slabs_ref.at[pl.ds(8 * j, 8), :],
            sems.at[j],
        )
        cp.start()
        copies.append(cp)
    lane = jax.lax.broadcasted_iota(jnp.int32, (1, _NB), 1)
    for j in range(cap):
        blk = idx_sref[r, j]
        start = jnp.minimum(blk * _NB, afloor)
        gcol_ref[pl.ds(j, 1), :] = jnp.where(
            lane < blk * _NB - start, jnp.int32(1 << 29), start + lane
        )
    for cp in copies:
        cp.wait()

    q = r % 8
    slabs = slabs_ref[...].reshape(cap, 8, _NB)
    picked = slabs[:, 0, :]
    for qq in range(1, 8):
        picked = jnp.where(q == qq, slabs[:, qq, :], picked)

    rowio = jax.lax.broadcasted_iota(jnp.int32, (nrows, _NB), 0)
    tail_row = jnp.max(
        jnp.where(rowio == r, tail_ref[...], -jnp.inf), axis=0, keepdims=True
    )
    subio = jax.lax.broadcasted_iota(jnp.int32, (8, _NB), 0)
    lane8 = jax.lax.broadcasted_iota(jnp.int32, (8, _NB), 1)
    tvals = jnp.where(subio == 0, jnp.broadcast_to(tail_row, (8, _NB)), -jnp.inf)
    tbase = ncols - _NB
    tgc = jnp.where(
        (subio == 0) & (tbase + lane8 >= afloor + _NB),
        tbase + lane8,
        jnp.int32(1 << 29),
    )

    gc = jnp.concatenate([gcol_ref[...], tgc], axis=0)
    vals = jnp.where(gc >= ncols, -jnp.inf, jnp.concatenate([picked, tvals], axis=0))
    rowmax = jnp.max(vals)

    b = jax.lax.bitcast_convert_type(vals, jnp.uint32)
    hi = jnp.uint32(0x80000000)
    s = jnp.where((b & hi) != 0, ~b, b | hi)

    def bbody(i, t):
        t_try = t | (hi >> i.astype(jnp.uint32))
        cnt = jnp.sum((s >= t_try).astype(jnp.int32))
        return jnp.where(cnt >= _K, t_try, t)

    t_bits = jax.lax.fori_loop(0, 32, bbody, jnp.uint32(0))
    tb = jnp.where((t_bits & hi) != 0, t_bits & ~hi, ~t_bits)
    thresh = jax.lax.bitcast_convert_type(tb, jnp.float32)

    keep = vals >= thresh
    z = jnp.sum(jnp.where(keep, jnp.exp(vals - rowmax), 0.0))

    p = (r * ncols + gc).astype(jnp.uint32)
    g = _gumbel_from_bits(_threefry_bits(p))
    cand = jnp.where(keep, vals + g, -jnp.inf)
    m2 = jnp.max(cand)
    wcol = jnp.min(jnp.where(cand == m2, gc, jnp.int32(1 << 30)))

    rowio = jax.lax.broadcasted_iota(jnp.int32, (nrows, 128), 0)
    colio = jax.lax.broadcasted_iota(jnp.int32, (nrows, 128), 1)
    vec = jnp.where(
        colio == 0,
        thresh,
        jnp.where(
            colio == 1,
            z,
            jnp.where(
                colio == 2,
                rowmax,
                jnp.where(colio == 3, wcol.astype(jnp.float32), 0.0),
            ),
        ),
    )
    prev = jnp.where(r == 0, jnp.zeros_like(vec), out_ref[...])
    out_ref[...] = jnp.where(rowio == r, vec, prev)


def _pass_c_kernel(x_ref, par_ref, out_ref, *, ncols):
    c = pl.program_id(1)
    x = x_ref[...]
    rows = x.shape[0]
    thresh = par_ref[:, 0:1]
    z = par_ref[:, 1:2]
    rowmax = par_ref[:, 2:3]
    lane = jax.lax.broadcasted_iota(jnp.int32, x.shape, 1)
    gcol = c * _W + lane
    keep = (x >= thresh) & (gcol < ncols)
    out_ref[...] = jnp.where(keep, jnp.exp(x - rowmax) / z, 0.0)


def kernel(logits, top_k):
    del top_k
    nrows, ncols = logits.shape
    ncb = math.ceil(ncols / _NB)
    cap = min(_CAP, ncb)
    nwide = math.ceil(ncols / _W)
    nsub = _W // _NB
    rblk = min(8, nrows)

    bmax = pl.pallas_call(
        functools.partial(_pass_a_kernel, ncols=ncols, ncb=ncb, nsub=nsub),
        grid=(nrows // rblk, nwide),
        in_specs=[pl.BlockSpec((rblk, _W), lambda r, c: (r, c))],
        out_specs=pl.BlockSpec((rblk, ncb), lambda r, c: (r, 0)),
        out_shape=jax.ShapeDtypeStruct((nrows, ncb), jnp.float32),
    )(logits)

    topidx = pl.pallas_call(
        functools.partial(_pass_a2_kernel, cap=cap),
        grid=(1,),
        in_specs=[pl.BlockSpec((nrows, ncb), lambda i: (0, 0))],
        out_specs=pl.BlockSpec((nrows, cap), lambda i: (0, 0)),
        out_shape=jax.ShapeDtypeStruct((nrows, cap), jnp.int32),
    )(bmax)

    params = pl.pallas_call(
        functools.partial(_pass_b_kernel, nrows=nrows, ncols=ncols, cap=cap),
        grid_spec=pltpu.PrefetchScalarGridSpec(
            num_scalar_prefetch=1,
            grid=(nrows,),
            in_specs=[
                pl.BlockSpec(memory_space=pl.ANY),
                pl.BlockSpec((nrows, _NB), lambda r, idx: (0, 0)),
            ],
            out_specs=pl.BlockSpec((nrows, 128), lambda r, idx: (0, 0)),
            scratch_shapes=[
                pltpu.VMEM((cap * 8, _NB), jnp.float32),
                pltpu.VMEM((cap, _NB), jnp.int32),
                pltpu.SemaphoreType.DMA((cap,)),
            ],
        ),
        out_shape=jax.ShapeDtypeStruct((nrows, 128), jnp.float32),
    )(topidx, logits, jax.lax.slice(logits, (0, ncols - _NB), (nrows, ncols)))

    probs = pl.pallas_call(
        functools.partial(_pass_c_kernel, ncols=ncols),
        grid=(nrows // rblk, nwide),
        in_specs=[
            pl.BlockSpec((rblk, _W), lambda r, c: (r, c)),
            pl.BlockSpec((rblk, 128), lambda r, c: (r, 0)),
        ],
        out_specs=pl.BlockSpec((rblk, _W), lambda r, c: (r, c)),
        out_shape=jax.ShapeDtypeStruct((nrows, ncols), jnp.float32),
    )(logits, params)

    idx_next = params[:, 3].astype(jnp.int32)
    return probs, idx_next


# speedup vs baseline: 69.9657x; 1.0179x over previous
import functools
import math

import jax
import jax.numpy as jnp
from jax.experimental import pallas as pl
from jax.experimental.pallas import tpu as pltpu

_NB = 1024
_W = 65536
_CAP = 64
_K = 50
_TINY = float(jnp.finfo(jnp.float32).tiny)


def _threefry_bits(x1):
    k1 = jnp.uint32(0)
    k2 = jnp.uint32(1)
    ks2 = jnp.uint32(0x1BD11BDA) ^ k1 ^ k2
    ks = (k1, k2, ks2)
    rot = ((13, 15, 26, 6), (17, 29, 16, 24))
    x0 = jnp.zeros_like(x1) + k1
    x1 = x1 + k2
    for i in range(5):
        for r in rot[i % 2]:
            x0 = x0 + x1
            x1 = (x1 << jnp.uint32(r)) | (x1 >> jnp.uint32(32 - r))
            x1 = x1 ^ x0
        x0 = x0 + ks[(i + 1) % 3]
        x1 = x1 + ks[(i + 2) % 3] + jnp.uint32(i + 1)
    return x0 ^ x1


def _gumbel_from_bits(bits):
    fb = (bits >> jnp.uint32(9)) | jnp.uint32(0x3F800000)
    f = jax.lax.bitcast_convert_type(fb, jnp.float32) - jnp.float32(1.0)
    tiny = jnp.float32(_TINY)
    u = jax.lax.max(tiny, f * (jnp.float32(1.0) - tiny) + tiny)
    return -jnp.log(-jnp.log(u))


def _pass_a_kernel(x_ref, bmax_ref, *, ncols, ncb, nsub):
    c = pl.program_id(1)
    x = x_ref[...]
    rows = x.shape[0]
    cio = jax.lax.broadcasted_iota(jnp.int32, (rows, ncb), 1)
    lane = jax.lax.broadcasted_iota(jnp.int32, (rows, _NB), 1)
    acc = bmax_ref[...]
    for i in range(nsub):
        sub = x[:, i * _NB : (i + 1) * _NB]
        gcol = c * _W + i * _NB + lane
        sub = jnp.where(gcol >= ncols, -jnp.inf, sub)
        m = jnp.max(sub, axis=1, keepdims=True)
        acc = jnp.where(cio == (c * nsub + i), m, acc)
    bmax_ref[...] = acc


def _pass_a2_kernel(bmax_ref, idx_ref, *, cap):
    bm = bmax_ref[...]
    rows, ncb = bm.shape
    cio = jax.lax.broadcasted_iota(jnp.int32, bm.shape, 1)
    capio = jax.lax.broadcasted_iota(jnp.int32, (rows, cap), 1)

    def body(t, carry):
        bm, acc = carry
        m = jnp.max(bm, axis=1, keepdims=True)
        pos = jnp.min(
            jnp.where(bm == m, cio, jnp.int32(1 << 30)), axis=1, keepdims=True
        )
        acc = jnp.where(capio == t, pos, acc)
        bm = jnp.where(cio == pos, -jnp.inf, bm)
        return bm, acc

    _, acc = jax.lax.fori_loop(0, cap, body, (bm, jnp.zeros((rows, cap), jnp.int32)))
    idx_ref[...] = acc


def _pass_b_kernel(
    idx_sref, x_hbm, tail_ref, out_ref, slabs_ref, gcol_ref, sems, *, nrows, ncols, cap
):
    r = pl.program_id(0)
    afloor = ((ncols - _NB) // 128) * 128
    rbase = pl.multiple_of((r // 8) * 8, 8)
    copies = []
    for j in range(cap):
        blk = idx_sref[r, j]
        start = pl.multiple_of(jnp.minimum(blk * _NB, afloor), 128)
        cp = pltpu.make_async_copy(
            x_hbm.at[pl.ds(rbase, 8), pl.ds(start, _NB)],
            slabs_ref.at[pl.ds(8 * j, 8), :],
            sems.at[j],
        )
        cp.start()
        copies.append(cp)
    lane = jax.lax.broadcasted_iota(jnp.int32, (1, _NB), 1)
    for j in range(cap):
        blk = idx_sref[r, j]
        start = jnp.minimum(blk * _NB, afloor)
        gcol_ref[pl.ds(j, 1), :] = jnp.where(
            lane < blk * _NB - start, jnp.int32(1 << 29), start + lane
        )
    for cp in copies:
        cp.wait()

    q = r % 8
    slabs = slabs_ref[...].reshape(cap, 8, _NB)
    picked = slabs[:, 0, :]
    for qq in range(1, 8):
        picked = jnp.where(q == qq, slabs[:, qq, :], picked)

    rowio = jax.lax.broadcasted_iota(jnp.int32, (nrows, _NB), 0)
    tail_row = jnp.max(
        jnp.where(rowio == r, tail_ref[...], -jnp.inf), axis=0, keepdims=True
    )
    subio = jax.lax.broadcasted_iota(jnp.int32, (8, _NB), 0)
    lane8 = jax.lax.broadcasted_iota(jnp.int32, (8, _NB), 1)
    tvals = jnp.where(subio == 0, jnp.broadcast_to(tail_row, (8, _NB)), -jnp.inf)
    tbase = ncols - _NB
    tgc = jnp.where(
        (subio == 0) & (tbase + lane8 >= afloor + _NB),
        tbase + lane8,
        jnp.int32(1 << 29),
    )

    gc = jnp.concatenate([gcol_ref[...], tgc], axis=0)
    vals = jnp.where(gc >= ncols, -jnp.inf, jnp.concatenate([picked, tvals], axis=0))
    rowmax = jnp.max(vals)

    b = jax.lax.bitcast_convert_type(vals, jnp.uint32)
    hi = jnp.uint32(0x80000000)
    s = jnp.where((b & hi) != 0, ~b, b | hi)

    def bbody(i, t):
        t_try = t | (hi >> i.astype(jnp.uint32))
        cnt = jnp.sum((s >= t_try).astype(jnp.int32))
        return jnp.where(cnt >= _K, t_try, t)

    t_bits = jax.lax.fori_loop(0, 32, bbody, jnp.uint32(0))
    tb = jnp.where((t_bits & hi) != 0, t_bits & ~hi, ~t_bits)
    thresh = jax.lax.bitcast_convert_type(tb, jnp.float32)

    keep = vals >= thresh
    z = jnp.sum(jnp.where(keep, jnp.exp(vals - rowmax), 0.0))

    p = (r * ncols + gc).astype(jnp.uint32)
    g = _gumbel_from_bits(_threefry_bits(p))
    cand = jnp.where(keep, vals + g, -jnp.inf)
    m2 = jnp.max(cand)
    wcol = jnp.min(jnp.where(cand == m2, gc, jnp.int32(1 << 30)))

    rowio = jax.lax.broadcasted_iota(jnp.int32, (nrows, 128), 0)
    colio = jax.lax.broadcasted_iota(jnp.int32, (nrows, 128), 1)
    vec = jnp.where(
        colio == 0,
        thresh,
        jnp.where(
            colio == 1,
            z,
            jnp.where(
                colio == 2,
                rowmax,
                jnp.where(colio == 3, wcol.astype(jnp.float32), 0.0),
            ),
        ),
    )
    prev = jnp.where(r == 0, jnp.zeros_like(vec), out_ref[...])
    out_ref[...] = jnp.where(rowio == r, vec, prev)


def _pass_c_kernel(x_ref, par_ref, out_ref, *, ncols, nwide):
    c = pl.program_id(1)
    x = x_ref[...]
    thresh = par_ref[:, 0:1]
    invz = 1.0 / par_ref[:, 1:2]
    rowmax = par_ref[:, 2:3]

    @pl.when(c < nwide - 1)
    def _():
        out_ref[...] = jnp.where(x >= thresh, jnp.exp(x - rowmax) * invz, 0.0)

    @pl.when(c == nwide - 1)
    def _():
        lane = jax.lax.broadcasted_iota(jnp.int32, x.shape, 1)
        keep = (x >= thresh) & (c * _W + lane < ncols)
        out_ref[...] = jnp.where(keep, jnp.exp(x - rowmax) * invz, 0.0)


def kernel(logits, top_k):
    del top_k
    nrows, ncols = logits.shape
    ncb = math.ceil(ncols / _NB)
    cap = min(_CAP, ncb)
    nwide = math.ceil(ncols / _W)
    nsub = _W // _NB
    rblk = min(8, nrows)

    bmax = pl.pallas_call(
        functools.partial(_pass_a_kernel, ncols=ncols, ncb=ncb, nsub=nsub),
        grid=(nrows // rblk, nwide),
        in_specs=[pl.BlockSpec((rblk, _W), lambda r, c: (r, c))],
        out_specs=pl.BlockSpec((rblk, ncb), lambda r, c: (r, 0)),
        out_shape=jax.ShapeDtypeStruct((nrows, ncb), jnp.float32),
    )(logits)

    topidx = pl.pallas_call(
        functools.partial(_pass_a2_kernel, cap=cap),
        grid=(1,),
        in_specs=[pl.BlockSpec((nrows, ncb), lambda i: (0, 0))],
        out_specs=pl.BlockSpec((nrows, cap), lambda i: (0, 0)),
        out_shape=jax.ShapeDtypeStruct((nrows, cap), jnp.int32),
    )(bmax)

    params = pl.pallas_call(
        functools.partial(_pass_b_kernel, nrows=nrows, ncols=ncols, cap=cap),
        grid_spec=pltpu.PrefetchScalarGridSpec(
            num_scalar_prefetch=1,
            grid=(nrows,),
            in_specs=[
                pl.BlockSpec(memory_space=pl.ANY),
                pl.BlockSpec((nrows, _NB), lambda r, idx: (0, 0)),
            ],
            out_specs=pl.BlockSpec((nrows, 128), lambda r, idx: (0, 0)),
            scratch_shapes=[
                pltpu.VMEM((cap * 8, _NB), jnp.float32),
                pltpu.VMEM((cap, _NB), jnp.int32),
                pltpu.SemaphoreType.DMA((cap,)),
            ],
        ),
        out_shape=jax.ShapeDtypeStruct((nrows, 128), jnp.float32),
    )(topidx, logits, jax.lax.slice(logits, (0, ncols - _NB), (nrows, ncols)))

    probs = pl.pallas_call(
        functools.partial(_pass_c_kernel, ncols=ncols, nwide=nwide),
        grid=(nrows // rblk, nwide),
        in_specs=[
            pl.BlockSpec((rblk, _W), lambda r, c: (r, c)),
            pl.BlockSpec((rblk, 128), lambda r, c: (r, 0)),
        ],
        out_specs=pl.BlockSpec((rblk, _W), lambda r, c: (r, c)),
        out_shape=jax.ShapeDtypeStruct((nrows, ncols), jnp.float32),
    )(logits, params)

    idx_next = params[:, 3].astype(jnp.int32)
    return probs, idx_next


# speedup vs baseline: 76.6638x; 1.0957x over previous
import functools
import math

import jax
import jax.numpy as jnp
from jax.experimental import pallas as pl
from jax.experimental.pallas import tpu as pltpu

_NB = 1024
_W = 262144
_CAP = 64
_K = 50
_TINY = float(jnp.finfo(jnp.float32).tiny)


def _threefry_bits(x1):
    k1 = jnp.uint32(0)
    k2 = jnp.uint32(1)
    ks2 = jnp.uint32(0x1BD11BDA) ^ k1 ^ k2
    ks = (k1, k2, ks2)
    rot = ((13, 15, 26, 6), (17, 29, 16, 24))
    x0 = jnp.zeros_like(x1) + k1
    x1 = x1 + k2
    for i in range(5):
        for r in rot[i % 2]:
            x0 = x0 + x1
            x1 = (x1 << jnp.uint32(r)) | (x1 >> jnp.uint32(32 - r))
            x1 = x1 ^ x0
        x0 = x0 + ks[(i + 1) % 3]
        x1 = x1 + ks[(i + 2) % 3] + jnp.uint32(i + 1)
    return x0 ^ x1


def _gumbel_from_bits(bits):
    fb = (bits >> jnp.uint32(9)) | jnp.uint32(0x3F800000)
    f = jax.lax.bitcast_convert_type(fb, jnp.float32) - jnp.float32(1.0)
    tiny = jnp.float32(_TINY)
    u = jax.lax.max(tiny, f * (jnp.float32(1.0) - tiny) + tiny)
    return -jnp.log(-jnp.log(u))


def _pass_a_kernel(x_ref, bmax_ref, *, ncols, ncb, nsub):
    c = pl.program_id(1)
    x = x_ref[...]
    rows = x.shape[0]
    cio = jax.lax.broadcasted_iota(jnp.int32, (rows, ncb), 1)
    lane = jax.lax.broadcasted_iota(jnp.int32, (rows, _NB), 1)
    acc = bmax_ref[...]
    for i in range(nsub):
        sub = x[:, i * _NB : (i + 1) * _NB]
        gcol = c * _W + i * _NB + lane
        sub = jnp.where(gcol >= ncols, -jnp.inf, sub)
        m = jnp.max(sub, axis=1, keepdims=True)
        acc = jnp.where(cio == (c * nsub + i), m, acc)
    bmax_ref[...] = acc


def _pass_a2_kernel(bmax_ref, idx_ref, *, cap):
    bm = bmax_ref[...]
    rows, ncb = bm.shape
    cio = jax.lax.broadcasted_iota(jnp.int32, bm.shape, 1)
    capio = jax.lax.broadcasted_iota(jnp.int32, (rows, cap), 1)

    def body(t, carry):
        bm, acc = carry
        m = jnp.max(bm, axis=1, keepdims=True)
        pos = jnp.min(
            jnp.where(bm == m, cio, jnp.int32(1 << 30)), axis=1, keepdims=True
        )
        acc = jnp.where(capio == t, pos, acc)
        bm = jnp.where(cio == pos, -jnp.inf, bm)
        return bm, acc

    _, acc = jax.lax.fori_loop(0, cap, body, (bm, jnp.zeros((rows, cap), jnp.int32)))
    idx_ref[...] = acc


def _pass_b_kernel(
    idx_sref, x_hbm, tail_ref, out_ref, slabs_ref, gcol_ref, sems, *, nrows, ncols, cap
):
    r = pl.program_id(0)
    afloor = ((ncols - _NB) // 128) * 128
    rbase = pl.multiple_of((r // 8) * 8, 8)
    copies = []
    for j in range(cap):
        blk = idx_sref[r, j]
        start = pl.multiple_of(jnp.minimum(blk * _NB, afloor), 128)
        cp = pltpu.make_async_copy(
            x_hbm.at[pl.ds(rbase, 8), pl.ds(start, _NB)],
            slabs_ref.at[pl.ds(8 * j, 8), :],
            sems.at[j],
        )
        cp.start()
        copies.append(cp)
    lane = jax.lax.broadcasted_iota(jnp.int32, (1, _NB), 1)
    for j in range(cap):
        blk = idx_sref[r, j]
        start = jnp.minimum(blk * _NB, afloor)
        gcol_ref[pl.ds(j, 1), :] = jnp.where(
            lane < blk * _NB - start, jnp.int32(1 << 29), start + lane
        )
    for cp in copies:
        cp.wait()

    q = r % 8
    slabs = slabs_ref[...].reshape(cap, 8, _NB)
    picked = slabs[:, 0, :]
    for qq in range(1, 8):
        picked = jnp.where(q == qq, slabs[:, qq, :], picked)

    rowio = jax.lax.broadcasted_iota(jnp.int32, (nrows, _NB), 0)
    tail_row = jnp.max(
        jnp.where(rowio == r, tail_ref[...], -jnp.inf), axis=0, keepdims=True
    )
    subio = jax.lax.broadcasted_iota(jnp.int32, (8, _NB), 0)
    lane8 = jax.lax.broadcasted_iota(jnp.int32, (8, _NB), 1)
    tvals = jnp.where(subio == 0, jnp.broadcast_to(tail_row, (8, _NB)), -jnp.inf)
    tbase = ncols - _NB
    tgc = jnp.where(
        (subio == 0) & (tbase + lane8 >= afloor + _NB),
        tbase + lane8,
        jnp.int32(1 << 29),
    )

    gc = jnp.concatenate([gcol_ref[...], tgc], axis=0)
    vals = jnp.where(gc >= ncols, -jnp.inf, jnp.concatenate([picked, tvals], axis=0))
    rowmax = jnp.max(vals)

    b = jax.lax.bitcast_convert_type(vals, jnp.uint32)
    hi = jnp.uint32(0x80000000)
    s = jnp.where((b & hi) != 0, ~b, b | hi)

    def bbody(i, t):
        t_try = t | (hi >> i.astype(jnp.uint32))
        cnt = jnp.sum((s >= t_try).astype(jnp.int32))
        return jnp.where(cnt >= _K, t_try, t)

    t_bits = jax.lax.fori_loop(0, 32, bbody, jnp.uint32(0))
    tb = jnp.where((t_bits & hi) != 0, t_bits & ~hi, ~t_bits)
    thresh = jax.lax.bitcast_convert_type(tb, jnp.float32)

    keep = vals >= thresh
    z = jnp.sum(jnp.where(keep, jnp.exp(vals - rowmax), 0.0))

    p = (r * ncols + gc).astype(jnp.uint32)
    g = _gumbel_from_bits(_threefry_bits(p))
    cand = jnp.where(keep, vals + g, -jnp.inf)
    m2 = jnp.max(cand)
    wcol = jnp.min(jnp.where(cand == m2, gc, jnp.int32(1 << 30)))

    rowio = jax.lax.broadcasted_iota(jnp.int32, (nrows, 128), 0)
    colio = jax.lax.broadcasted_iota(jnp.int32, (nrows, 128), 1)
    vec = jnp.where(
        colio == 0,
        thresh,
        jnp.where(
            colio == 1,
            z,
            jnp.where(
                colio == 2,
                rowmax,
                jnp.where(colio == 3, wcol.astype(jnp.float32), 0.0),
            ),
        ),
    )
    prev = jnp.where(r == 0, jnp.zeros_like(vec), out_ref[...])
    out_ref[...] = jnp.where(rowio == r, vec, prev)


def _pass_c_kernel(x_ref, par_ref, out_ref, *, ncols, nwide):
    c = pl.program_id(1)
    x = x_ref[...]
    thresh = par_ref[:, 0:1]
    invz = 1.0 / par_ref[:, 1:2]
    rowmax = par_ref[:, 2:3]

    @pl.when(c < nwide - 1)
    def _():
        out_ref[...] = jnp.where(x >= thresh, jnp.exp(x - rowmax) * invz, 0.0)

    @pl.when(c == nwide - 1)
    def _():
        lane = jax.lax.broadcasted_iota(jnp.int32, x.shape, 1)
        keep = (x >= thresh) & (c * _W + lane < ncols)
        out_ref[...] = jnp.where(keep, jnp.exp(x - rowmax) * invz, 0.0)


def kernel(logits, top_k):
    del top_k
    nrows, ncols = logits.shape
    ncb = math.ceil(ncols / _NB)
    cap = min(_CAP, ncb)
    nwide = math.ceil(ncols / _W)
    nsub = _W // _NB
    rblk = min(8, nrows)

    bmax = pl.pallas_call(
        functools.partial(_pass_a_kernel, ncols=ncols, ncb=ncb, nsub=nsub),
        grid=(nrows // rblk, nwide),
        in_specs=[pl.BlockSpec((rblk, _W), lambda r, c: (r, c))],
        out_specs=pl.BlockSpec((rblk, ncb), lambda r, c: (r, 0)),
        out_shape=jax.ShapeDtypeStruct((nrows, ncb), jnp.float32),
    )(logits)

    topidx = pl.pallas_call(
        functools.partial(_pass_a2_kernel, cap=cap),
        grid=(1,),
        in_specs=[pl.BlockSpec((nrows, ncb), lambda i: (0, 0))],
        out_specs=pl.BlockSpec((nrows, cap), lambda i: (0, 0)),
        out_shape=jax.ShapeDtypeStruct((nrows, cap), jnp.int32),
    )(bmax)

    params = pl.pallas_call(
        functools.partial(_pass_b_kernel, nrows=nrows, ncols=ncols, cap=cap),
        grid_spec=pltpu.PrefetchScalarGridSpec(
            num_scalar_prefetch=1,
            grid=(nrows,),
            in_specs=[
                pl.BlockSpec(memory_space=pl.ANY),
                pl.BlockSpec((nrows, _NB), lambda r, idx: (0, 0)),
            ],
            out_specs=pl.BlockSpec((nrows, 128), lambda r, idx: (0, 0)),
            scratch_shapes=[
                pltpu.VMEM((cap * 8, _NB), jnp.float32),
                pltpu.VMEM((cap, _NB), jnp.int32),
                pltpu.SemaphoreType.DMA((cap,)),
            ],
        ),
        out_shape=jax.ShapeDtypeStruct((nrows, 128), jnp.float32),
    )(topidx, logits, jax.lax.slice(logits, (0, ncols - _NB), (nrows, ncols)))

    probs = pl.pallas_call(
        functools.partial(_pass_c_kernel, ncols=ncols, nwide=nwide),
        grid=(nrows // rblk, nwide),
        in_specs=[
            pl.BlockSpec((rblk, _W), lambda r, c: (r, c)),
            pl.BlockSpec((rblk, 128), lambda r, c: (r, 0)),
        ],
        out_specs=pl.BlockSpec((rblk, _W), lambda r, c: (r, c)),
        out_shape=jax.ShapeDtypeStruct((nrows, ncols), jnp.float32),
    )(logits, params)

    idx_next = params[:, 3].astype(jnp.int32)
    return probs, idx_next
